# baseline (speedup 1.0000x reference)
import functools

import jax
import jax.numpy as jnp
from jax import lax
from jax.experimental import pallas as pl
from jax.experimental.pallas import tpu as pltpu
from jax.experimental.pallas import tpu_sc as plsc

_K = 16
_R2 = 0.25 * 0.25
_B, _N, _C = 2, 4096, 3
_LANES = 16
_NCHUNK = _N // _LANES
_NW = 32
_QPW = (_B * _N) // _NW


def _ball_qloss_sc(xs, fs):
    mesh = plsc.VectorSubcoreMesh(core_axis_name="c", subcore_axis_name="s")

    @functools.partial(
        pl.kernel,
        mesh=mesh,
        compiler_params=pltpu.CompilerParams(needs_layout_passes=False),
        out_type=jax.ShapeDtypeStruct((_NW, _LANES), jnp.float32),
        scratch_types=[
            pltpu.VMEM((3 * _N,), jnp.float32),
            pltpu.VMEM((3 * _N,), jnp.float32),
            pltpu.VMEM((_LANES,), jnp.int32),
            pltpu.VMEM((_LANES,), jnp.float32),
        ],
    )
    def k(xs_hbm, fs_hbm, out_hbm, x_v, f_v, idx_v, acc_v):
        w = lax.axis_index("s") * 2 + lax.axis_index("c")
        b = w // 16
        qbase = (w % 16) * _QPW
        pltpu.sync_copy(xs_hbm.at[pl.ds(b * 3 * _N, 3 * _N)], x_v)
        pltpu.sync_copy(fs_hbm.at[pl.ds(b * 3 * _N, 3 * _N)], f_v)
        idx_v[...] = jnp.zeros((_LANES,), jnp.int32)
        lanes = lax.iota(jnp.int32, _LANES)

        def qbody(q, acc):
            qi = qbase + q
            qiv = jnp.full((_LANES,), qi, jnp.int32)
            xi = plsc.load_gather(x_v, [qiv])
            yi = plsc.load_gather(x_v, [qiv + _N])
            zi = plsc.load_gather(x_v, [qiv + 2 * _N])

            def cond(st):
                cntv, ch = st
                return jnp.all(cntv < _K) & (ch < _NCHUNK)

            def body(st):
                cntv, ch = st
                j0 = ch * _LANES
                dx = x_v[pl.ds(j0, _LANES)] - xi
                dy = x_v[pl.ds(_N + j0, _LANES)] - yi
                dz = x_v[pl.ds(2 * _N + j0, _LANES)] - zi
                d2 = dx * dx + dy * dy + dz * dz
                m = d2 < _R2
                mi = m.astype(jnp.int32)
                rank = cntv + plsc.cumsum(mi)
                sel = m & (rank <= _K)
                pos = jnp.clip(rank - 1, 0, _LANES - 1)
                plsc.store_scatter(idx_v, [pos], j0 + lanes, mask=sel)
                return cntv + plsc.all_reduce_population_count(m), ch + 1

            cnt, _ = lax.while_loop(
                cond, body, (jnp.zeros((_LANES,), jnp.int32), jnp.int32(0)))

            ib = idx_v[...]
            first = plsc.load_gather(idx_v, [jnp.zeros((_LANES,), jnp.int32)])
            idxv = jnp.where(lanes < cnt, ib, first)
            dfx = plsc.load_gather(f_v, [idxv]) - plsc.load_gather(f_v, [qiv])
            dfy = plsc.load_gather(f_v, [idxv + _N]) - plsc.load_gather(
                f_v, [qiv + _N])
            dfz = plsc.load_gather(f_v, [idxv + 2 * _N]) - plsc.load_gather(
                f_v, [qiv + 2 * _N])
            n2 = dfx * dfx + dfy * dfy + dfz * dfz
            iy = lax.bitcast_convert_type(n2, jnp.int32)
            iy = jnp.int32(0x5F3759DF) - lax.shift_right_logical(iy, 1)
            y = lax.bitcast_convert_type(iy, jnp.float32)
            for _ in range(3):
                y = y * (1.5 - 0.5 * (n2 * y) * y)
            return acc + n2 * y

        acc = lax.fori_loop(0, _QPW, qbody, jnp.zeros((_LANES,), jnp.float32))
        acc_v[...] = acc
        pltpu.sync_copy(acc_v, out_hbm.at[w])

    return k(xs, fs)


def kernel(pc_source, pred_flow):
    xs = jnp.transpose(pc_source, (0, 2, 1)).reshape(-1)
    fs = jnp.transpose(pred_flow, (0, 2, 1)).reshape(-1)
    out = _ball_qloss_sc(xs, fs)
    return jnp.sum(out) / (_B * _N * _K)


# speedup vs baseline: 1.1583x; 1.1583x over previous
import functools

import jax
import jax.numpy as jnp
from jax import lax
from jax.experimental import pallas as pl
from jax.experimental.pallas import tpu as pltpu
from jax.experimental.pallas import tpu_sc as plsc

_K = 16
_R2 = 0.25 * 0.25
_B, _N, _C = 2, 4096, 3
_LANES = 16
_NCHUNK = _N // _LANES
_NW = 32
_QPW = (_B * _N) // _NW


def _ball_qloss_sc(xs, fs):
    mesh = plsc.VectorSubcoreMesh(core_axis_name="c", subcore_axis_name="s")

    @functools.partial(
        pl.kernel,
        mesh=mesh,
        compiler_params=pltpu.CompilerParams(needs_layout_passes=False),
        out_type=jax.ShapeDtypeStruct((_NW, _LANES), jnp.float32),
        scratch_types=[
            pltpu.VMEM((3 * _N,), jnp.float32),
            pltpu.VMEM((3 * _N,), jnp.float32),
            pltpu.VMEM((_LANES,), jnp.int32),
            pltpu.VMEM((_LANES,), jnp.float32),
        ],
    )
    def k(xs_hbm, fs_hbm, out_hbm, x_v, f_v, idx_v, acc_v):
        w = lax.axis_index("s") * 2 + lax.axis_index("c")
        b = w // 16
        qbase = (w % 16) * _QPW
        pltpu.sync_copy(xs_hbm.at[pl.ds(b * 3 * _N, 3 * _N)], x_v)
        pltpu.sync_copy(fs_hbm.at[pl.ds(b * 3 * _N, 3 * _N)], f_v)
        idx_v[...] = jnp.zeros((_LANES,), jnp.int32)
        lanes = lax.iota(jnp.int32, _LANES)

        def qbody(q, acc):
            qi = qbase + q
            qiv = jnp.full((_LANES,), qi, jnp.int32)
            xi = plsc.load_gather(x_v, [qiv])
            yi = plsc.load_gather(x_v, [qiv + _N])
            zi = plsc.load_gather(x_v, [qiv + 2 * _N])

            def cond(st):
                cnt, ch = st
                return (cnt < _K) & (ch < _NCHUNK)

            def body(st):
                cnt, ch = st
                j0 = ch * _LANES
                dx = x_v[pl.ds(j0, _LANES)] - xi
                dy = x_v[pl.ds(_N + j0, _LANES)] - yi
                dz = x_v[pl.ds(2 * _N + j0, _LANES)] - zi
                d2 = dx * dx + dy * dy + dz * dz
                m = d2 < _R2
                mi = m.astype(jnp.int32)
                cs = plsc.cumsum(mi)
                rank = cnt + cs
                sel = m & (rank <= _K)
                pos = jnp.clip(rank - 1, 0, _LANES - 1)
                plsc.store_scatter(idx_v, [pos], j0 + lanes, mask=sel)
                return cnt + cs[_LANES - 1], ch + 1

            cnt, _ = lax.while_loop(cond, body, (jnp.int32(0), jnp.int32(0)))

            ib = idx_v[...]
            first = plsc.load_gather(idx_v, [jnp.zeros((_LANES,), jnp.int32)])
            idxv = jnp.where(lanes < cnt, ib, first)
            dfx = plsc.load_gather(f_v, [idxv]) - plsc.load_gather(f_v, [qiv])
            dfy = plsc.load_gather(f_v, [idxv + _N]) - plsc.load_gather(
                f_v, [qiv + _N])
            dfz = plsc.load_gather(f_v, [idxv + 2 * _N]) - plsc.load_gather(
                f_v, [qiv + 2 * _N])
            n2 = dfx * dfx + dfy * dfy + dfz * dfz
            iy = lax.bitcast_convert_type(n2, jnp.int32)
            iy = jnp.int32(0x5F3759DF) - lax.shift_right_logical(iy, 1)
            y = lax.bitcast_convert_type(iy, jnp.float32)
            for _ in range(3):
                y = y * (1.5 - 0.5 * (n2 * y) * y)
            return acc + n2 * y

        acc = lax.fori_loop(0, _QPW, qbody, jnp.zeros((_LANES,), jnp.float32))
        acc_v[...] = acc
        pltpu.sync_copy(acc_v, out_hbm.at[w])

    return k(xs, fs)


def kernel(pc_source, pred_flow):
    xs = jnp.transpose(pc_source, (0, 2, 1)).reshape(-1)
    fs = jnp.transpose(pred_flow, (0, 2, 1)).reshape(-1)
    out = _ball_qloss_sc(xs, fs)
    return jnp.sum(out) / (_B * _N * _K)


# speedup vs baseline: 1.3815x; 1.1928x over previous
import functools

import jax
import jax.numpy as jnp
from jax import lax
from jax.experimental import pallas as pl
from jax.experimental.pallas import tpu as pltpu
from jax.experimental.pallas import tpu_sc as plsc

_K = 16
_R2 = 0.25 * 0.25
_B, _N, _C = 2, 4096, 3
_LANES = 16
_NCHUNK = _N // _LANES
_NW = 32
_QPW = (_B * _N) // _NW
_U = 4


def _ball_qloss_sc(xs, fs):
    mesh = plsc.VectorSubcoreMesh(core_axis_name="c", subcore_axis_name="s")

    @functools.partial(
        pl.kernel,
        mesh=mesh,
        compiler_params=pltpu.CompilerParams(needs_layout_passes=False),
        out_type=jax.ShapeDtypeStruct((_NW, _LANES), jnp.float32),
        scratch_types=[
            pltpu.VMEM((3 * _N,), jnp.float32),
            pltpu.VMEM((3 * _N,), jnp.float32),
            pltpu.VMEM((_LANES,), jnp.int32),
            pltpu.VMEM((_LANES,), jnp.float32),
        ],
    )
    def k(xs_hbm, fs_hbm, out_hbm, x_v, f_v, idx_v, acc_v):
        w = lax.axis_index("s") * 2 + lax.axis_index("c")
        b = w // 16
        qbase = (w % 16) * _QPW
        pltpu.sync_copy(xs_hbm.at[pl.ds(b * 3 * _N, 3 * _N)], x_v)
        pltpu.sync_copy(fs_hbm.at[pl.ds(b * 3 * _N, 3 * _N)], f_v)
        idx_v[...] = jnp.zeros((_LANES,), jnp.int32)
        lanes = lax.iota(jnp.int32, _LANES)

        def qbody(q, acc):
            qi = qbase + q
            qiv = jnp.full((_LANES,), qi, jnp.int32)
            xi = plsc.load_gather(x_v, [qiv])
            yi = plsc.load_gather(x_v, [qiv + _N])
            zi = plsc.load_gather(x_v, [qiv + 2 * _N])

            def cond(st):
                cnt, ch = st
                return (cnt < _K) & (ch < _NCHUNK // _U)

            def body(st):
                cnt, ch = st
                base = ch * (_LANES * _U)
                for u in range(_U):
                    j0 = base + u * _LANES
                    dx = x_v[pl.ds(j0, _LANES)] - xi
                    dy = x_v[pl.ds(_N + j0, _LANES)] - yi
                    dz = x_v[pl.ds(2 * _N + j0, _LANES)] - zi
                    d2 = dx * dx + dy * dy + dz * dz
                    m = d2 < _R2
                    cs = plsc.cumsum(m.astype(jnp.int32))
                    rank = cnt + cs
                    sel = m & (rank <= _K)
                    pos = jnp.clip(rank - 1, 0, _LANES - 1)
                    plsc.store_scatter(idx_v, [pos], j0 + lanes, mask=sel)
                    cnt = cnt + cs[_LANES - 1]
                return cnt, ch + 1

            cnt, _ = lax.while_loop(cond, body, (jnp.int32(0), jnp.int32(0)))

            ib = idx_v[...]
            first = plsc.load_gather(idx_v, [jnp.zeros((_LANES,), jnp.int32)])
            idxv = jnp.where(lanes < cnt, ib, first)
            dfx = plsc.load_gather(f_v, [idxv]) - plsc.load_gather(f_v, [qiv])
            dfy = plsc.load_gather(f_v, [idxv + _N]) - plsc.load_gather(
                f_v, [qiv + _N])
            dfz = plsc.load_gather(f_v, [idxv + 2 * _N]) - plsc.load_gather(
                f_v, [qiv + 2 * _N])
            n2 = dfx * dfx + dfy * dfy + dfz * dfz
            iy = lax.bitcast_convert_type(n2, jnp.int32)
            iy = jnp.int32(0x5F3759DF) - lax.shift_right_logical(iy, 1)
            y = lax.bitcast_convert_type(iy, jnp.float32)
            for _ in range(3):
                y = y * (1.5 - 0.5 * (n2 * y) * y)
            return acc + n2 * y

        acc = lax.fori_loop(0, _QPW, qbody, jnp.zeros((_LANES,), jnp.float32))
        acc_v[...] = acc
        pltpu.sync_copy(acc_v, out_hbm.at[w])

    return k(xs, fs)


def kernel(pc_source, pred_flow):
    xs = jnp.transpose(pc_source, (0, 2, 1)).reshape(-1)
    fs = jnp.transpose(pred_flow, (0, 2, 1)).reshape(-1)
    out = _ball_qloss_sc(xs, fs)
    return jnp.sum(out) / (_B * _N * _K)


# speedup vs baseline: 1.7824x; 1.2901x over previous
import functools

import jax
import jax.numpy as jnp
from jax import lax
from jax.experimental import pallas as pl
from jax.experimental.pallas import tpu as pltpu
from jax.experimental.pallas import tpu_sc as plsc

_K = 16
_R2 = 0.25 * 0.25
_B, _N, _C = 2, 4096, 3
_LANES = 16
_NCHUNK = _N // _LANES
_NW = 32
_QPW = (_B * _N) // _NW
_U = 4


def _ball_qloss_sc(xs, fs):
    mesh = plsc.VectorSubcoreMesh(core_axis_name="c", subcore_axis_name="s")

    @functools.partial(
        pl.kernel,
        mesh=mesh,
        compiler_params=pltpu.CompilerParams(needs_layout_passes=False),
        out_type=jax.ShapeDtypeStruct((_NW, _LANES), jnp.float32),
        scratch_types=[
            pltpu.VMEM((3 * _N,), jnp.float32),
            pltpu.VMEM((3 * _N,), jnp.float32),
            pltpu.VMEM((128,), jnp.int32),
            pltpu.VMEM((_LANES,), jnp.float32),
        ],
    )
    def k(xs_hbm, fs_hbm, out_hbm, x_v, f_v, idx_v, acc_v):
        w = lax.axis_index("s") * 2 + lax.axis_index("c")
        b = w // 16
        qbase = (w % 16) * _QPW
        pltpu.sync_copy(xs_hbm.at[pl.ds(b * 3 * _N, 3 * _N)], x_v)
        pltpu.sync_copy(fs_hbm.at[pl.ds(b * 3 * _N, 3 * _N)], f_v)
        for z in range(0, 128, _LANES):
            idx_v[pl.ds(z, _LANES)] = jnp.zeros((_LANES,), jnp.int32)
        lanes = lax.iota(jnp.int32, _LANES)

        def qbody(q, acc):
            qi = qbase + q
            qiv = jnp.full((_LANES,), qi, jnp.int32)
            xi = plsc.load_gather(x_v, [qiv])
            yi = plsc.load_gather(x_v, [qiv + _N])
            zi = plsc.load_gather(x_v, [qiv + 2 * _N])

            def cond(st):
                cnt, ch = st
                return (cnt < _K) & (ch < _NCHUNK // _U)

            def body(st):
                cnt, ch = st
                base = ch * (_LANES * _U)
                for u in range(_U):
                    j0 = base + u * _LANES
                    dx = x_v[pl.ds(j0, _LANES)] - xi
                    dy = x_v[pl.ds(_N + j0, _LANES)] - yi
                    dz = x_v[pl.ds(2 * _N + j0, _LANES)] - zi
                    d2 = dx * dx + dy * dy + dz * dz
                    m = d2 < _R2
                    plsc.store_compressed(
                        idx_v.at[pl.ds(cnt, _LANES)], j0 + lanes, mask=m)
                    pc = plsc.all_reduce_population_count(m)
                    cnt = cnt + pc[0]
                return cnt, ch + 1

            cnt, _ = lax.while_loop(cond, body, (jnp.int32(0), jnp.int32(0)))

            ib = idx_v[pl.ds(0, _LANES)]
            first = plsc.load_gather(idx_v, [jnp.zeros((_LANES,), jnp.int32)])
            idxv = jnp.where(lanes < cnt, ib, first)
            dfx = plsc.load_gather(f_v, [idxv]) - plsc.load_gather(f_v, [qiv])
            dfy = plsc.load_gather(f_v, [idxv + _N]) - plsc.load_gather(
                f_v, [qiv + _N])
            dfz = plsc.load_gather(f_v, [idxv + 2 * _N]) - plsc.load_gather(
                f_v, [qiv + 2 * _N])
            n2 = dfx * dfx + dfy * dfy + dfz * dfz
            iy = lax.bitcast_convert_type(n2, jnp.int32)
            iy = jnp.int32(0x5F3759DF) - lax.shift_right_logical(iy, 1)
            y = lax.bitcast_convert_type(iy, jnp.float32)
            for _ in range(3):
                y = y * (1.5 - 0.5 * (n2 * y) * y)
            return acc + n2 * y

        acc = lax.fori_loop(0, _QPW, qbody, jnp.zeros((_LANES,), jnp.float32))
        acc_v[...] = acc
        pltpu.sync_copy(acc_v, out_hbm.at[w])

    return k(xs, fs)


def kernel(pc_source, pred_flow):
    xs = jnp.transpose(pc_source, (0, 2, 1)).reshape(-1)
    fs = jnp.transpose(pred_flow, (0, 2, 1)).reshape(-1)
    out = _ball_qloss_sc(xs, fs)
    return jnp.sum(out) / (_B * _N * _K)
